# baseline (speedup 1.0000x reference)
import jax
import jax.numpy as jnp
from jax.experimental import pallas as pl

_F32 = jnp.float32
_BF16 = jnp.bfloat16


def _xw_kernel(*refs):
    out_ref = refs[-1]
    nin = (len(refs) - 1) // 2
    acc = None
    for i in range(nin):
        f = refs[i][...]
        w = refs[nin + i][...].astype(_BF16)
        p = jnp.dot(f, w, preferred_element_type=_F32)
        acc = p if acc is None else acc + p
    out_ref[...] = acc.astype(_BF16)


def _pass1_kernel(adj_ref, t0_ref, b0_ref, adjbf_ref, h0_ref):
    abf = adj_ref[...].astype(_BF16)
    adjbf_ref[...] = abf
    p0 = jnp.dot(abf, t0_ref[...], preferred_element_type=_F32) + b0_ref[...]
    h0_ref[...] = jnp.tanh(p0).astype(_BF16)


def _pass2_kernel(adjbf_ref, t1_ref, b1_ref, h1_ref):
    p1 = (jnp.dot(adjbf_ref[...], t1_ref[...], preferred_element_type=_F32)
          + b1_ref[...])
    h1_ref[...] = jnp.tanh(p1).astype(_BF16)


def _pass3_kernel(adjbf_ref, t2_ref, bo_ref, out_ref):
    logits = (jnp.dot(adjbf_ref[...], t2_ref[...], preferred_element_type=_F32)
              + bo_ref[...])
    m = jnp.max(logits, axis=1, keepdims=True)
    shifted = logits - m
    out_ref[...] = shifted - jnp.log(
        jnp.sum(jnp.exp(shifted), axis=1, keepdims=True))


def _xw(feats, weights, out_w):
    n = feats[0].shape[0]
    specs = [pl.BlockSpec(f.shape, lambda: (0, 0)) for f in feats]
    specs += [pl.BlockSpec(w.shape, lambda: (0, 0)) for w in weights]
    return pl.pallas_call(
        _xw_kernel,
        in_specs=specs,
        out_specs=pl.BlockSpec((n, out_w), lambda: (0, 0)),
        out_shape=jax.ShapeDtypeStruct((n, out_w), _BF16),
    )(*feats, *weights)


def kernel(x, adj, W0, b0, W1, b1, W_out, b_out):
    n, nf = x.shape
    nh = W0.shape[1]
    nc = W_out.shape[1]

    bm = 256 if n >= 256 else n
    grid = (pl.cdiv(n, bm),)

    x_bf = x.astype(_BF16)
    W1a, W1b = W1[:nf], W1[nf:]
    Woa, Wob, Woc = W_out[:nf], W_out[nf:nf + nh], W_out[nf + nh:]
    b0r = b0.reshape(1, nh)
    b1r = b1.reshape(1, nh)
    bor = b_out.reshape(1, nc)

    row_spec = pl.BlockSpec((bm, n), lambda i: (i, 0))

    def full(a):
        return pl.BlockSpec(a.shape, lambda i: (0,) * a.ndim)

    def act_spec(w):
        return pl.BlockSpec((bm, w), lambda i: (i, 0))

    t0 = _xw([x_bf], [W0], nh)
    adj_bf, h0 = pl.pallas_call(
        _pass1_kernel,
        grid=grid,
        in_specs=[row_spec, full(t0), full(b0r)],
        out_specs=[row_spec, act_spec(nh)],
        out_shape=[jax.ShapeDtypeStruct((n, n), _BF16),
                   jax.ShapeDtypeStruct((n, nh), _BF16)],
    )(adj, t0, b0r)

    t1 = _xw([x_bf, h0], [W1a, W1b], nh)
    h1 = pl.pallas_call(
        _pass2_kernel,
        grid=grid,
        in_specs=[row_spec, full(t1), full(b1r)],
        out_specs=act_spec(nh),
        out_shape=jax.ShapeDtypeStruct((n, nh), _BF16),
    )(adj_bf, t1, b1r)

    t2 = _xw([x_bf, h0, h1], [Woa, Wob, Woc], nc)
    logp = pl.pallas_call(
        _pass3_kernel,
        grid=grid,
        in_specs=[row_spec, full(t2), full(bor)],
        out_specs=act_spec(nc),
        out_shape=jax.ShapeDtypeStruct((n, nc), _F32),
    )(adj_bf, t2, bor)

    return logp


# speedup vs baseline: 1.0926x; 1.0926x over previous
import jax
import jax.numpy as jnp
from jax.experimental import pallas as pl
from jax.experimental.pallas import tpu as pltpu

_F32 = jnp.float32
_BF16 = jnp.bfloat16


def _prep_t(i, feat_refs, w_refs, t_ref):
    @pl.when(i == 0)
    def _():
        acc = None
        for f_ref, w_ref in zip(feat_refs, w_refs):
            p = jnp.dot(f_ref[...], w_ref[...].astype(_BF16),
                        preferred_element_type=_F32)
            acc = p if acc is None else acc + p
        t_ref[...] = acc.astype(_BF16)


def _pass1_kernel(adj_ref, x_ref, W0_ref, b0_ref, adjbf_ref, h0_ref, t_ref):
    _prep_t(pl.program_id(0), [x_ref], [W0_ref], t_ref)
    abf = adj_ref[...].astype(_BF16)
    adjbf_ref[...] = abf
    p0 = jnp.dot(abf, t_ref[...], preferred_element_type=_F32) + b0_ref[...]
    h0_ref[...] = jnp.tanh(p0).astype(_BF16)


def _pass2_kernel(adjbf_ref, x_ref, h0_ref, W1a_ref, W1b_ref, b1_ref,
                  h1_ref, t_ref):
    _prep_t(pl.program_id(0), [x_ref, h0_ref], [W1a_ref, W1b_ref], t_ref)
    p1 = (jnp.dot(adjbf_ref[...], t_ref[...], preferred_element_type=_F32)
          + b1_ref[...])
    h1_ref[...] = jnp.tanh(p1).astype(_BF16)


def _pass3_kernel(adjbf_ref, x_ref, h0_ref, h1_ref,
                  Woa_ref, Wob_ref, Woc_ref, bo_ref, out_ref, t_ref):
    _prep_t(pl.program_id(0), [x_ref, h0_ref, h1_ref],
            [Woa_ref, Wob_ref, Woc_ref], t_ref)
    logits = (jnp.dot(adjbf_ref[...], t_ref[...], preferred_element_type=_F32)
              + bo_ref[...])
    m = jnp.max(logits, axis=1, keepdims=True)
    shifted = logits - m
    out_ref[...] = shifted - jnp.log(
        jnp.sum(jnp.exp(shifted), axis=1, keepdims=True))


def kernel(x, adj, W0, b0, W1, b1, W_out, b_out):
    n, nf = x.shape
    nh = W0.shape[1]
    nc = W_out.shape[1]

    if n >= 512:
        bm1, bm2 = 256, 512
    else:
        bm1 = bm2 = min(256, n)
    grid1 = (pl.cdiv(n, bm1),)
    grid2 = (pl.cdiv(n, bm2),)

    x_bf = x.astype(_BF16)
    W1a, W1b = W1[:nf], W1[nf:]
    Woa, Wob, Woc = W_out[:nf], W_out[nf:nf + nh], W_out[nf + nh:]
    b0r = b0.reshape(1, nh)
    b1r = b1.reshape(1, nh)
    bor = b_out.reshape(1, nc)

    def row_spec(bm):
        return pl.BlockSpec((bm, n), lambda i: (i, 0))

    def full(a):
        return pl.BlockSpec(a.shape, lambda i: (0,) * a.ndim)

    def act_spec(bm, w):
        return pl.BlockSpec((bm, w), lambda i: (i, 0))

    adj_bf, h0 = pl.pallas_call(
        _pass1_kernel,
        grid=grid1,
        in_specs=[row_spec(bm1), full(x_bf), full(W0), full(b0r)],
        out_specs=[row_spec(bm1), act_spec(bm1, nh)],
        out_shape=[jax.ShapeDtypeStruct((n, n), _BF16),
                   jax.ShapeDtypeStruct((n, nh), _BF16)],
        scratch_shapes=[pltpu.VMEM((n, nh), _BF16)],
    )(adj, x_bf, W0, b0r)

    h1 = pl.pallas_call(
        _pass2_kernel,
        grid=grid2,
        in_specs=[row_spec(bm2), full(x_bf), full(h0), full(W1a), full(W1b),
                  full(b1r)],
        out_specs=act_spec(bm2, nh),
        out_shape=jax.ShapeDtypeStruct((n, nh), _BF16),
        scratch_shapes=[pltpu.VMEM((n, nh), _BF16)],
    )(adj_bf, x_bf, h0, W1a, W1b, b1r)

    logp = pl.pallas_call(
        _pass3_kernel,
        grid=grid2,
        in_specs=[row_spec(bm2), full(x_bf), full(h0), full(h1),
                  full(Woa), full(Wob), full(Woc), full(bor)],
        out_specs=act_spec(bm2, nc),
        out_shape=jax.ShapeDtypeStruct((n, nc), _F32),
        scratch_shapes=[pltpu.VMEM((n, nc), _BF16)],
    )(adj_bf, x_bf, h0, h1, Woa, Wob, Woc, bor)

    return logp


# speedup vs baseline: 1.1248x; 1.0295x over previous
import jax
import jax.numpy as jnp
from jax.experimental import pallas as pl
from jax.experimental.pallas import tpu as pltpu

_F32 = jnp.float32
_CP = pltpu.CompilerParams(vmem_limit_bytes=110 * 1024 * 1024)
_BF16 = jnp.bfloat16


def _prep_t(i, feat_refs, w_refs, t_ref):
    @pl.when(i == 0)
    def _():
        acc = None
        for f_ref, w_ref in zip(feat_refs, w_refs):
            p = jnp.dot(f_ref[...], w_ref[...].astype(_BF16),
                        preferred_element_type=_F32)
            acc = p if acc is None else acc + p
        t_ref[...] = acc.astype(_BF16)


def _pass1_kernel(adj_ref, x_ref, W0_ref, b0_ref, adjbf_ref, h0_ref, t_ref):
    _prep_t(pl.program_id(0), [x_ref], [W0_ref], t_ref)
    abf = adj_ref[...].astype(_BF16)
    adjbf_ref[...] = abf
    p0 = jnp.dot(abf, t_ref[...], preferred_element_type=_F32) + b0_ref[...]
    h0_ref[...] = jnp.tanh(p0).astype(_BF16)


def _pass2_kernel(adjbf_ref, x_ref, h0_ref, W1a_ref, W1b_ref, b1_ref,
                  h1_ref, t_ref):
    _prep_t(pl.program_id(0), [x_ref, h0_ref], [W1a_ref, W1b_ref], t_ref)
    p1 = (jnp.dot(adjbf_ref[...], t_ref[...], preferred_element_type=_F32)
          + b1_ref[...])
    h1_ref[...] = jnp.tanh(p1).astype(_BF16)


def _pass3_kernel(adjbf_ref, x_ref, h0_ref, h1_ref,
                  Woa_ref, Wob_ref, Woc_ref, bo_ref, out_ref, t_ref):
    _prep_t(pl.program_id(0), [x_ref, h0_ref, h1_ref],
            [Woa_ref, Wob_ref, Woc_ref], t_ref)
    logits = (jnp.dot(adjbf_ref[...], t_ref[...], preferred_element_type=_F32)
              + bo_ref[...])
    m = jnp.max(logits, axis=1, keepdims=True)
    shifted = logits - m
    out_ref[...] = shifted - jnp.log(
        jnp.sum(jnp.exp(shifted), axis=1, keepdims=True))


def kernel(x, adj, W0, b0, W1, b1, W_out, b_out):
    n, nf = x.shape
    nh = W0.shape[1]
    nc = W_out.shape[1]

    if n >= 1024:
        bm1, bm2 = 384, 1024
    else:
        bm1 = bm2 = min(256, n)
    grid1 = (pl.cdiv(n, bm1),)
    grid2 = (pl.cdiv(n, bm2),)

    x_bf = x.astype(_BF16)
    W1a, W1b = W1[:nf], W1[nf:]
    Woa, Wob, Woc = W_out[:nf], W_out[nf:nf + nh], W_out[nf + nh:]
    b0r = b0.reshape(1, nh)
    b1r = b1.reshape(1, nh)
    bor = b_out.reshape(1, nc)

    def row_spec(bm):
        return pl.BlockSpec((bm, n), lambda i: (i, 0))

    def full(a):
        return pl.BlockSpec(a.shape, lambda i: (0,) * a.ndim)

    def act_spec(bm, w):
        return pl.BlockSpec((bm, w), lambda i: (i, 0))

    adj_bf, h0 = pl.pallas_call(
        _pass1_kernel,
        grid=grid1,
        in_specs=[row_spec(bm1), full(x_bf), full(W0), full(b0r)],
        out_specs=[row_spec(bm1), act_spec(bm1, nh)],
        out_shape=[jax.ShapeDtypeStruct((n, n), _BF16),
                   jax.ShapeDtypeStruct((n, nh), _BF16)],
        scratch_shapes=[pltpu.VMEM((n, nh), _BF16)],
        compiler_params=_CP,
    )(adj, x_bf, W0, b0r)

    h1 = pl.pallas_call(
        _pass2_kernel,
        grid=grid2,
        in_specs=[row_spec(bm2), full(x_bf), full(h0), full(W1a), full(W1b),
                  full(b1r)],
        out_specs=act_spec(bm2, nh),
        out_shape=jax.ShapeDtypeStruct((n, nh), _BF16),
        scratch_shapes=[pltpu.VMEM((n, nh), _BF16)],
        compiler_params=_CP,
    )(adj_bf, x_bf, h0, W1a, W1b, b1r)

    logp = pl.pallas_call(
        _pass3_kernel,
        grid=grid2,
        in_specs=[row_spec(bm2), full(x_bf), full(h0), full(h1),
                  full(Woa), full(Wob), full(Woc), full(bor)],
        out_specs=act_spec(bm2, nc),
        out_shape=jax.ShapeDtypeStruct((n, nc), _F32),
        scratch_shapes=[pltpu.VMEM((n, nc), _BF16)],
        compiler_params=_CP,
    )(adj_bf, x_bf, h0, h1, Woa, Wob, Woc, bor)

    return logp
